# baseline (speedup 1.0000x reference)
import functools

import jax
import jax.numpy as jnp
from jax import lax
from jax.experimental import pallas as pl
from jax.experimental.pallas import tpu as pltpu

try:
    from jax.experimental.pallas import tpu_sc as plsc
except ImportError:
    plsc = None

_TN = 4608
_TK = 512


def _codes_body(x_ref, e_hbm, out_ref, e_ref, e2_ref, sem, *, n_rows, k_total):
    @pl.when(pl.program_id(0) == 0)
    def _():
        pltpu.make_async_copy(e_hbm, e_ref, sem).start()
        pltpu.make_async_copy(e_hbm, e_ref, sem).wait()
        for j in range(k_total // _TK):
            ec = e_ref[j * _TK:(j + 1) * _TK, :]
            e2_ref[j:j + 1, :] = jnp.sum(ec * ec, axis=1).reshape(1, _TK)

    x = x_ref[...]
    x2 = jnp.sum(x * x, axis=1, keepdims=True)
    vm = jnp.full((n_rows, _TK), jnp.inf, dtype=jnp.float32)
    vidx = jnp.zeros((n_rows, _TK), dtype=jnp.float32)
    def _xe2(j):
        e = e_ref[j * _TK:(j + 1) * _TK, :]
        return lax.dot_general(x, e * (-2.0),
                               (((1,), (1,)), ((), ())),
                               preferred_element_type=jnp.float32)

    n_chunks = k_total // _TK
    xe2 = _xe2(0)
    for j in range(n_chunks):
        xe2_cur = xe2
        if j + 1 < n_chunks:
            xe2 = _xe2(j + 1)
        e2 = e2_ref[j:j + 1, :]
        s = (x2 + e2) + xe2_cur
        upd = s < vm
        vidx = jnp.where(upd, float(j), vidx)
        vm = jnp.minimum(vm, s)
    lanef = lax.broadcasted_iota(jnp.int32, (n_rows, _TK), 1).astype(jnp.float32)
    combined = vidx * float(_TK) + lanef
    lmin = jnp.min(vm, axis=1, keepdims=True)
    cand = jnp.where(vm == lmin, combined, float(k_total))
    a = jnp.min(cand, axis=1, keepdims=True)
    out_ref[...] = a.astype(jnp.int32)


def _codes_tc(x, embed, *, interpret=False):
    n, d = x.shape
    k, _ = embed.shape
    grid = (n // _TN,)
    return pl.pallas_call(
        functools.partial(_codes_body, n_rows=_TN, k_total=k),
        grid=grid,
        in_specs=[
            pl.BlockSpec((_TN, d), lambda i: (i, 0)),
            pl.BlockSpec(memory_space=pl.ANY),
        ],
        out_specs=pl.BlockSpec((_TN, 1), lambda i: (i, 0)),
        out_shape=jax.ShapeDtypeStruct((n, 1), jnp.int32),
        scratch_shapes=[
            pltpu.VMEM((k, d), jnp.float32),
            pltpu.VMEM((k // _TK, _TK), jnp.float32),
            pltpu.SemaphoreType.DMA,
        ],
        interpret=interpret,
    )(x, embed)


def _gather_sc(embed, codes):
    k, d = embed.shape
    n = codes.shape[0]
    info = plsc.get_sparse_core_info()
    nc, ns = info.num_cores, info.num_subcores
    nw = nc * ns
    b_per_w = n // nw
    ch = 96
    nch = b_per_w // ch
    mesh = plsc.VectorSubcoreMesh(core_axis_name="c", subcore_axis_name="s")

    @functools.partial(
        pl.kernel,
        mesh=mesh,
        out_type=jax.ShapeDtypeStruct((n, d), jnp.float32),
        scratch_types=[
            pltpu.VMEM((b_per_w,), jnp.int32),
            pltpu.VMEM((b_per_w, d), jnp.float32),
            pltpu.SemaphoreType.DMA,
        ],
    )
    def gather_kernel(table_hbm, idx_hbm, out_hbm, idx_v, rows_v, sem):
        wid = lax.axis_index("s") * nc + lax.axis_index("c")
        base = wid * b_per_w
        pltpu.sync_copy(idx_hbm.at[pl.ds(base, b_per_w)], idx_v)
        cps = []
        for c in range(nch):
            cps.append(pltpu.async_copy(
                table_hbm.at[idx_v.at[pl.ds(c * ch, ch)]],
                rows_v.at[pl.ds(c * ch, ch)], sem))
        for cp in cps:
            cp.wait()
        pltpu.sync_copy(rows_v, out_hbm.at[pl.ds(base, b_per_w)])

    return gather_kernel(embed, codes)


def kernel(input, embed):
    b, n, d = input.shape
    x = input.reshape(b * n, d)
    codes = _codes_tc(x, embed).reshape(b * n)
    quantize = _gather_sc(embed, codes)
    return quantize.reshape(b, n, d), codes.reshape(b, n)


# speedup vs baseline: 1.2254x; 1.2254x over previous
import functools

import jax
import jax.numpy as jnp
from jax import lax
from jax.experimental import pallas as pl
from jax.experimental.pallas import tpu as pltpu

try:
    from jax.experimental.pallas import tpu_sc as plsc
except ImportError:
    plsc = None

_TN = 2304
_TK = 512


def _codes_body(x_ref, e_hbm, out_ref, e_ref, e2_ref, sem, *, n_rows, k_total):
    @pl.when(pl.program_id(0) == 0)
    def _():
        pltpu.make_async_copy(e_hbm, e_ref, sem).start()
        pltpu.make_async_copy(e_hbm, e_ref, sem).wait()
        for j in range(k_total // _TK):
            ec = e_ref[j * _TK:(j + 1) * _TK, :]
            e2_ref[j:j + 1, :] = jnp.sum(ec * ec, axis=1).reshape(1, _TK)

    x = x_ref[...]
    x2 = jnp.sum(x * x, axis=1, keepdims=True)
    vm = jnp.full((n_rows, _TK), jnp.inf, dtype=jnp.float32)
    vidx = jnp.zeros((n_rows, _TK), dtype=jnp.float32)
    def _xe2(j):
        e = e_ref[j * _TK:(j + 1) * _TK, :]
        return lax.dot_general(x, e * (-2.0),
                               (((1,), (1,)), ((), ())),
                               preferred_element_type=jnp.float32)

    n_chunks = k_total // _TK
    xe2 = _xe2(0)
    for j in range(n_chunks):
        xe2_cur = xe2
        if j + 1 < n_chunks:
            xe2 = _xe2(j + 1)
        e2 = e2_ref[j:j + 1, :]
        s = (x2 + e2) + xe2_cur
        upd = s < vm
        vidx = jnp.where(upd, float(j), vidx)
        vm = jnp.minimum(vm, s)
    lanef = lax.broadcasted_iota(jnp.int32, (n_rows, _TK), 1).astype(jnp.float32)
    combined = vidx * float(_TK) + lanef
    lmin = jnp.min(vm, axis=1, keepdims=True)
    cand = jnp.where(vm == lmin, combined, float(k_total))
    a = jnp.min(cand, axis=1, keepdims=True)
    out_ref[...] = a.astype(jnp.int32)


def _codes_tc(x, embed, *, interpret=False):
    n, d = x.shape
    k, _ = embed.shape
    grid = (n // _TN,)
    return pl.pallas_call(
        functools.partial(_codes_body, n_rows=_TN, k_total=k),
        grid=grid,
        in_specs=[
            pl.BlockSpec((_TN, d), lambda i: (i, 0)),
            pl.BlockSpec(memory_space=pl.ANY),
        ],
        out_specs=pl.BlockSpec((_TN, 1), lambda i: (i, 0)),
        out_shape=jax.ShapeDtypeStruct((n, 1), jnp.int32),
        scratch_shapes=[
            pltpu.VMEM((k, d), jnp.float32),
            pltpu.VMEM((k // _TK, _TK), jnp.float32),
            pltpu.SemaphoreType.DMA,
        ],
        interpret=interpret,
    )(x, embed)


def _gather_sc(embed, codes):
    k, d = embed.shape
    n = codes.shape[0]
    info = plsc.get_sparse_core_info()
    nc, ns = info.num_cores, info.num_subcores
    nw = nc * ns
    b_per_w = n // nw
    ch = 96
    nch = b_per_w // ch
    mesh = plsc.VectorSubcoreMesh(core_axis_name="c", subcore_axis_name="s")

    @functools.partial(
        pl.kernel,
        mesh=mesh,
        out_type=jax.ShapeDtypeStruct((n, d), jnp.float32),
        scratch_types=[
            pltpu.VMEM((b_per_w,), jnp.int32),
            pltpu.VMEM((b_per_w, d), jnp.float32),
            pltpu.SemaphoreType.DMA,
        ],
    )
    def gather_kernel(table_hbm, idx_hbm, out_hbm, idx_v, rows_v, sem):
        wid = lax.axis_index("s") * nc + lax.axis_index("c")
        base = wid * b_per_w
        pltpu.sync_copy(idx_hbm.at[pl.ds(base, b_per_w)], idx_v)
        cps = []
        for c in range(nch):
            cps.append(pltpu.async_copy(
                table_hbm.at[idx_v.at[pl.ds(c * ch, ch)]],
                rows_v.at[pl.ds(c * ch, ch)], sem))
        for cp in cps:
            cp.wait()
        pltpu.sync_copy(rows_v, out_hbm.at[pl.ds(base, b_per_w)])

    return gather_kernel(embed, codes)


def kernel(input, embed):
    b, n, d = input.shape
    x = input.reshape(b * n, d)
    codes = _codes_tc(x, embed).reshape(b * n)
    quantize = _gather_sc(embed, codes)
    return quantize.reshape(b, n, d), codes.reshape(b, n)
